# baseline (speedup 1.0000x reference)
import jax
import jax.numpy as jnp
from jax.experimental import pallas as pl
from jax.experimental.pallas import tpu as pltpu


def _body(col_ref, row_ref, out_ref, pat_ref):
    w, d2 = col_ref.shape
    h = row_ref.shape[0]

    @pl.when(pl.program_id(0) == 0)
    def _():
        x_part = jnp.broadcast_to(col_ref[...][None, :, :], (h, w, d2))
        y_part = jnp.broadcast_to(row_ref[...][:, None, :], (h, w, d2))
        pat_ref[...] = jnp.concatenate([x_part, y_part], axis=-1)

    for j in range(out_ref.shape[0]):
        out_ref[j] = pat_ref[...]


def kernel(tensor, row_embed, col_embed):
    b = tensor.shape[0]
    h, w = tensor.shape[-2], tensor.shape[-1]
    d2 = row_embed.shape[-1]
    d = 2 * d2
    bb = 4 if b % 4 == 0 else (2 if b % 2 == 0 else 1)
    out = pl.pallas_call(
        _body,
        grid=(b // bb,),
        in_specs=[
            pl.BlockSpec((w, d2), lambda i: (0, 0)),
            pl.BlockSpec((h, d2), lambda i: (0, 0)),
        ],
        out_specs=pl.BlockSpec((bb, h, w, d), lambda i: (i, 0, 0, 0)),
        out_shape=jax.ShapeDtypeStruct((b, h, w, d), jnp.float32),
        scratch_shapes=[
            pltpu.VMEM((h, w, d), jnp.float32),
        ],
    )(col_embed, row_embed)
    return jnp.transpose(out, (0, 3, 1, 2))
